# baseline (speedup 1.0000x reference)
import functools

import jax
import jax.numpy as jnp
from jax import lax
from jax.experimental import pallas as pl
from jax.experimental.pallas import tpu as pltpu
from jax.experimental.pallas import tpu_sc as plsc

N_NODES = 10000
D_FEAT = 128
N_PAD = 10240
PAD_ROW = N_NODES
N_EDGES = 320000
BLK = 128
K_BLK = 80
E_PAD = K_BLK * BLK * 32
ROWS_PER_TILE = N_PAD // 16

_mesh = lambda: plsc.VectorSubcoreMesh(core_axis_name="c", subcore_axis_name="s")
_sc_params = lambda: pltpu.CompilerParams(use_tc_tiling_on_sc=False)


def _deg_kernel():
    @functools.partial(
        pl.kernel,
        mesh=_mesh(),
        out_type=jax.ShapeDtypeStruct((2 * N_PAD,), jnp.float32),
        compiler_params=_sc_params(),
        scratch_types=[
            pltpu.VMEM((K_BLK, BLK), jnp.int32),
            pltpu.VMEM((BLK,), jnp.float32),
            pltpu.VMEM((ROWS_PER_TILE,), jnp.float32),
            pltpu.VMEM_SHARED((N_PAD,), jnp.float32),
        ],
    )
    def degk(dst_hbm, out_hbm, dst_v, ones_v, zero_v, acc_sh):
        c = lax.axis_index("c")
        s = lax.axis_index("s")
        pltpu.sync_copy(dst_hbm.at[c * 16 + s], dst_v)
        ones16 = jnp.ones((16,), jnp.float32)
        zero16 = jnp.zeros((16,), jnp.float32)
        for i in range(BLK // 16):
            ones_v[pl.ds(i * 16, 16)] = ones16

        def zb(i, carry):
            zero_v[pl.ds(i * 16, 16)] = zero16
            return carry

        lax.fori_loop(0, ROWS_PER_TILE // 16, zb, 0)
        pltpu.sync_copy(zero_v, acc_sh.at[pl.ds(s * ROWS_PER_TILE, ROWS_PER_TILE)])
        plsc.subcore_barrier()

        def body(j, carry):
            pltpu.sync_copy(ones_v, acc_sh.at[dst_v.at[j]], add=True)
            return carry

        lax.fori_loop(0, K_BLK, body, 0)
        plsc.subcore_barrier()
        pltpu.sync_copy(
            acc_sh.at[pl.ds(s * ROWS_PER_TILE, ROWS_PER_TILE)],
            out_hbm.at[pl.ds(c * N_PAD + s * ROWS_PER_TILE, ROWS_PER_TILE)],
        )

    return degk


def _agg_kernel(d):
    @functools.partial(
        pl.kernel,
        mesh=_mesh(),
        out_type=jax.ShapeDtypeStruct((2, N_PAD, d), jnp.float32),
        compiler_params=_sc_params(),
        scratch_types=[
            pltpu.VMEM((K_BLK, BLK), jnp.int32),
            pltpu.VMEM((K_BLK, BLK), jnp.int32),
            pltpu.VMEM((BLK, d), jnp.float32),
            pltpu.VMEM((BLK, d), jnp.float32),
            pltpu.VMEM((ROWS_PER_TILE, d), jnp.float32),
            pltpu.VMEM_SHARED((N_PAD, d), jnp.float32),
            pltpu.SemaphoreType.DMA,
            pltpu.SemaphoreType.DMA,
            pltpu.SemaphoreType.DMA,
            pltpu.SemaphoreType.DMA,
        ],
    )
    def aggk(p_hbm, src_hbm, dst_hbm, out_hbm,
             src_v, dst_v, buf0, buf1, zero_v, acc_sh, gs0, gs1, ss0, ss1):
        c = lax.axis_index("c")
        s = lax.axis_index("s")
        wid = c * 16 + s
        pltpu.sync_copy(src_hbm.at[wid], src_v)
        pltpu.sync_copy(dst_hbm.at[wid], dst_v)
        zero16 = jnp.zeros((16,), jnp.float32)

        def zb(r, carry):
            for cc in range(d // 16):
                zero_v[r, pl.ds(cc * 16, 16)] = zero16
            return carry

        lax.fori_loop(0, ROWS_PER_TILE, zb, 0)
        pltpu.sync_copy(zero_v, acc_sh.at[pl.ds(s * ROWS_PER_TILE, ROWS_PER_TILE)])
        plsc.subcore_barrier()

        half = K_BLK // 2
        pltpu.async_copy(p_hbm.at[src_v.at[0]], buf0, gs0)

        def body(i, carry):
            j0 = 2 * i
            pltpu.make_async_copy(p_hbm.at[src_v.at[j0]], buf0, gs0).wait()

            @pl.when(i > 0)
            def _():
                pltpu.make_async_copy(buf1, acc_sh.at[dst_v.at[j0 - 1]], ss1).wait()

            pltpu.async_copy(p_hbm.at[src_v.at[j0 + 1]], buf1, gs1)
            pltpu.async_copy(buf0, acc_sh.at[dst_v.at[j0]], ss0, add=True)
            pltpu.make_async_copy(p_hbm.at[src_v.at[j0 + 1]], buf1, gs1).wait()
            pltpu.make_async_copy(buf0, acc_sh.at[dst_v.at[j0]], ss0).wait()

            @pl.when(i < half - 1)
            def _():
                pltpu.async_copy(p_hbm.at[src_v.at[j0 + 2]], buf0, gs0)

            pltpu.async_copy(buf1, acc_sh.at[dst_v.at[j0 + 1]], ss1, add=True)
            return carry

        lax.fori_loop(0, half, body, 0)
        pltpu.make_async_copy(buf1, acc_sh.at[dst_v.at[K_BLK - 1]], ss1).wait()
        plsc.subcore_barrier()
        pltpu.sync_copy(
            acc_sh.at[pl.ds(s * ROWS_PER_TILE, ROWS_PER_TILE)],
            out_hbm.at[c, pl.ds(s * ROWS_PER_TILE, ROWS_PER_TILE)],
        )

    return aggk


def _tc_stage_b(deg_ref, x_ref, w1_ref, p1_ref, dinv_ref):
    deg = deg_ref[0:1, :] + deg_ref[1:2, :] + 1.0
    dinv = jnp.transpose(lax.rsqrt(deg), (1, 0))
    dinv_ref[...] = dinv
    u = jnp.dot(x_ref[...], w1_ref[...], preferred_element_type=jnp.float32)
    p1_ref[pl.ds(0, N_NODES)] = u * dinv[:N_NODES]
    p1_ref[pl.ds(N_NODES, N_PAD - N_NODES)] = jnp.zeros(
        (N_PAD - N_NODES, 16), jnp.float32
    )


def _tc_stage_d(parts_ref, p1_ref, dinv_ref, b1_ref, w2_ref, p2_ref):
    dinv = dinv_ref[...]
    agg = parts_ref[0] + parts_ref[1] + p1_ref[...]
    h1 = jnp.maximum(agg * dinv + b1_ref[...], 0.0)
    p2 = jnp.dot(h1 * dinv, w2_ref[...], preferred_element_type=jnp.float32)
    p2_ref[pl.ds(0, N_NODES)] = p2[:N_NODES]
    p2_ref[pl.ds(N_NODES, N_PAD - N_NODES)] = jnp.zeros(
        (N_PAD - N_NODES, 32), jnp.float32
    )


def _tc_stage_f(parts_ref, p2_ref, dinv_ref, b2_ref, wout_ref, bout_ref, out_ref):
    dinv = dinv_ref[pl.ds(0, N_NODES)]
    agg = parts_ref[0, pl.ds(0, N_NODES)] + parts_ref[1, pl.ds(0, N_NODES)]
    agg = agg + p2_ref[pl.ds(0, N_NODES)]
    h2 = jnp.maximum(agg * dinv + b2_ref[...], 0.0)
    out = jnp.dot(h2, wout_ref[...], preferred_element_type=jnp.float32)
    out_ref[...] = (out + bout_ref[...]).reshape(N_NODES // 10, 10, D_FEAT)


def kernel(x, edge_index, W1, b1, W2, b2, W_out, b_out):
    e_pad = jnp.pad(
        edge_index, ((0, 0), (0, E_PAD - N_EDGES)), constant_values=PAD_ROW
    )
    src = e_pad[0].reshape(32, K_BLK, BLK)
    dst = e_pad[1].reshape(32, K_BLK, BLK)

    deg_flat = _deg_kernel()(dst)

    p1, dinv = pl.pallas_call(
        _tc_stage_b,
        out_shape=(
            jax.ShapeDtypeStruct((N_PAD, 16), jnp.float32),
            jax.ShapeDtypeStruct((N_PAD, 1), jnp.float32),
        ),
    )(deg_flat.reshape(2, N_PAD), x, W1)

    agg1 = _agg_kernel(16)(p1, src, dst)

    p2 = pl.pallas_call(
        _tc_stage_d,
        out_shape=jax.ShapeDtypeStruct((N_PAD, 32), jnp.float32),
    )(agg1, p1, dinv, b1, W2)

    agg2 = _agg_kernel(32)(p2, src, dst)

    return pl.pallas_call(
        _tc_stage_f,
        out_shape=jax.ShapeDtypeStruct((N_NODES // 10, 10, D_FEAT), jnp.float32),
    )(agg2, p2, dinv, b2, W_out, b_out)


# speedup vs baseline: 1.8988x; 1.8988x over previous
import functools

import jax
import jax.numpy as jnp
from jax import lax
from jax.experimental import pallas as pl
from jax.experimental.pallas import tpu as pltpu
from jax.experimental.pallas import tpu_sc as plsc

N_NODES = 10000
D_FEAT = 128
N_PAD = 10240
PAD_ROW = N_NODES
N_EDGES = 320000
BLK = 128
K_BLK = 80
E_PAD = K_BLK * BLK * 32
ROWS_PER_TILE = N_PAD // 16

_mesh = lambda: plsc.VectorSubcoreMesh(core_axis_name="c", subcore_axis_name="s")
_sc_params = lambda: pltpu.CompilerParams(use_tc_tiling_on_sc=False)


def _deg_kernel():
    @functools.partial(
        pl.kernel,
        mesh=_mesh(),
        out_type=jax.ShapeDtypeStruct((2 * N_PAD,), jnp.float32),
        compiler_params=_sc_params(),
        scratch_types=[
            pltpu.VMEM((K_BLK, BLK), jnp.int32),
            pltpu.VMEM((BLK,), jnp.float32),
            pltpu.VMEM((ROWS_PER_TILE,), jnp.float32),
            pltpu.VMEM_SHARED((N_PAD,), jnp.float32),
        ],
    )
    def degk(dst_hbm, out_hbm, dst_v, ones_v, zero_v, acc_sh):
        c = lax.axis_index("c")
        s = lax.axis_index("s")
        pltpu.sync_copy(dst_hbm.at[c * 16 + s], dst_v)
        ones16 = jnp.ones((16,), jnp.float32)
        zero16 = jnp.zeros((16,), jnp.float32)
        for i in range(BLK // 16):
            ones_v[pl.ds(i * 16, 16)] = ones16

        def zb(i, carry):
            zero_v[pl.ds(i * 16, 16)] = zero16
            return carry

        lax.fori_loop(0, ROWS_PER_TILE // 16, zb, 0)
        pltpu.sync_copy(zero_v, acc_sh.at[pl.ds(s * ROWS_PER_TILE, ROWS_PER_TILE)])
        plsc.subcore_barrier()

        def body(j, carry):
            pltpu.sync_copy(ones_v, acc_sh.at[dst_v.at[j]], add=True)
            return carry

        lax.fori_loop(0, K_BLK, body, 0)
        plsc.subcore_barrier()
        pltpu.sync_copy(
            acc_sh.at[pl.ds(s * ROWS_PER_TILE, ROWS_PER_TILE)],
            out_hbm.at[pl.ds(c * N_PAD + s * ROWS_PER_TILE, ROWS_PER_TILE)],
        )

    return degk


def _agg_kernel(d):
    @functools.partial(
        pl.kernel,
        mesh=_mesh(),
        out_type=jax.ShapeDtypeStruct((2, N_PAD, d), jnp.float32),
        compiler_params=_sc_params(),
        scratch_types=[
            pltpu.VMEM((K_BLK, BLK), jnp.int32),
            pltpu.VMEM((K_BLK, BLK), jnp.int32),
            pltpu.VMEM((BLK, d), jnp.float32),
            pltpu.VMEM((BLK, d), jnp.float32),
            pltpu.VMEM((ROWS_PER_TILE, d), jnp.float32),
            pltpu.VMEM_SHARED((N_PAD, d), jnp.float32),
            pltpu.VMEM_SHARED((N_PAD, d), jnp.float32),
            pltpu.SemaphoreType.DMA,
            pltpu.SemaphoreType.DMA,
            pltpu.SemaphoreType.DMA,
            pltpu.SemaphoreType.DMA,
        ],
    )
    def aggk(p_hbm, src_hbm, dst_hbm, out_hbm,
             src_v, dst_v, buf0, buf1, zero_v, acc_sh, p_sh, gs0, gs1, ss0, ss1):
        c = lax.axis_index("c")
        s = lax.axis_index("s")
        wid = c * 16 + s
        pltpu.sync_copy(src_hbm.at[wid], src_v)
        pltpu.sync_copy(dst_hbm.at[wid], dst_v)
        pltpu.sync_copy(
            p_hbm.at[pl.ds(s * ROWS_PER_TILE, ROWS_PER_TILE)],
            p_sh.at[pl.ds(s * ROWS_PER_TILE, ROWS_PER_TILE)],
        )
        zero16 = jnp.zeros((16,), jnp.float32)

        def zb(r, carry):
            for cc in range(d // 16):
                zero_v[r, pl.ds(cc * 16, 16)] = zero16
            return carry

        lax.fori_loop(0, ROWS_PER_TILE, zb, 0)
        pltpu.sync_copy(zero_v, acc_sh.at[pl.ds(s * ROWS_PER_TILE, ROWS_PER_TILE)])
        plsc.subcore_barrier()

        half = K_BLK // 2
        pltpu.async_copy(p_sh.at[src_v.at[0]], buf0, gs0)

        def body(i, carry):
            j0 = 2 * i
            pltpu.make_async_copy(p_sh.at[src_v.at[j0]], buf0, gs0).wait()

            @pl.when(i > 0)
            def _():
                pltpu.make_async_copy(buf1, acc_sh.at[dst_v.at[j0 - 1]], ss1).wait()

            pltpu.async_copy(p_sh.at[src_v.at[j0 + 1]], buf1, gs1)
            pltpu.async_copy(buf0, acc_sh.at[dst_v.at[j0]], ss0, add=True)
            pltpu.make_async_copy(p_sh.at[src_v.at[j0 + 1]], buf1, gs1).wait()
            pltpu.make_async_copy(buf0, acc_sh.at[dst_v.at[j0]], ss0).wait()

            @pl.when(i < half - 1)
            def _():
                pltpu.async_copy(p_sh.at[src_v.at[j0 + 2]], buf0, gs0)

            pltpu.async_copy(buf1, acc_sh.at[dst_v.at[j0 + 1]], ss1, add=True)
            return carry

        lax.fori_loop(0, half, body, 0)
        pltpu.make_async_copy(buf1, acc_sh.at[dst_v.at[K_BLK - 1]], ss1).wait()
        plsc.subcore_barrier()
        pltpu.sync_copy(
            acc_sh.at[pl.ds(s * ROWS_PER_TILE, ROWS_PER_TILE)],
            out_hbm.at[c, pl.ds(s * ROWS_PER_TILE, ROWS_PER_TILE)],
        )

    return aggk


def _tc_stage_b(deg_ref, x_ref, w1_ref, p1_ref, dinv_ref):
    deg = deg_ref[0:1, :] + deg_ref[1:2, :] + 1.0
    dinv = jnp.transpose(lax.rsqrt(deg), (1, 0))
    dinv_ref[...] = dinv
    u = jnp.dot(x_ref[...], w1_ref[...], preferred_element_type=jnp.float32)
    p1_ref[pl.ds(0, N_NODES)] = u * dinv[:N_NODES]
    p1_ref[pl.ds(N_NODES, N_PAD - N_NODES)] = jnp.zeros(
        (N_PAD - N_NODES, 16), jnp.float32
    )


def _tc_stage_d(parts_ref, p1_ref, dinv_ref, b1_ref, w2_ref, p2_ref):
    dinv = dinv_ref[...]
    agg = parts_ref[0] + parts_ref[1] + p1_ref[...]
    h1 = jnp.maximum(agg * dinv + b1_ref[...], 0.0)
    p2 = jnp.dot(h1 * dinv, w2_ref[...], preferred_element_type=jnp.float32)
    p2_ref[pl.ds(0, N_NODES)] = p2[:N_NODES]
    p2_ref[pl.ds(N_NODES, N_PAD - N_NODES)] = jnp.zeros(
        (N_PAD - N_NODES, 32), jnp.float32
    )


def _tc_stage_f(parts_ref, p2_ref, dinv_ref, b2_ref, wout_ref, bout_ref, out_ref):
    dinv = dinv_ref[pl.ds(0, N_NODES)]
    agg = parts_ref[0, pl.ds(0, N_NODES)] + parts_ref[1, pl.ds(0, N_NODES)]
    agg = agg + p2_ref[pl.ds(0, N_NODES)]
    h2 = jnp.maximum(agg * dinv + b2_ref[...], 0.0)
    out = jnp.dot(h2, wout_ref[...], preferred_element_type=jnp.float32)
    out_ref[...] = (out + bout_ref[...]).reshape(N_NODES // 10, 10, D_FEAT)


def kernel(x, edge_index, W1, b1, W2, b2, W_out, b_out):
    e_pad = jnp.pad(
        edge_index, ((0, 0), (0, E_PAD - N_EDGES)), constant_values=PAD_ROW
    )
    src = e_pad[0].reshape(32, K_BLK, BLK)
    dst = e_pad[1].reshape(32, K_BLK, BLK)

    deg_flat = _deg_kernel()(dst)

    p1, dinv = pl.pallas_call(
        _tc_stage_b,
        out_shape=(
            jax.ShapeDtypeStruct((N_PAD, 16), jnp.float32),
            jax.ShapeDtypeStruct((N_PAD, 1), jnp.float32),
        ),
    )(deg_flat.reshape(2, N_PAD), x, W1)

    agg1 = _agg_kernel(16)(p1, src, dst)

    p2 = pl.pallas_call(
        _tc_stage_d,
        out_shape=jax.ShapeDtypeStruct((N_PAD, 32), jnp.float32),
    )(agg1, p1, dinv, b1, W2)

    agg2 = _agg_kernel(32)(p2, src, dst)

    return pl.pallas_call(
        _tc_stage_f,
        out_shape=jax.ShapeDtypeStruct((N_NODES // 10, 10, D_FEAT), jnp.float32),
    )(agg2, p2, dinv, b2, W_out, b_out)


# speedup vs baseline: 1.9945x; 1.0504x over previous
import functools

import jax
import jax.numpy as jnp
from jax import lax
from jax.experimental import pallas as pl
from jax.experimental.pallas import tpu as pltpu
from jax.experimental.pallas import tpu_sc as plsc

N_NODES = 10000
D_FEAT = 128
N_PAD = 10240
PAD_ROW = N_NODES
N_EDGES = 320000
BLK = 128
K_BLK = 80
E_PAD = K_BLK * BLK * 32
ROWS_PER_TILE = N_PAD // 16

_mesh = lambda: plsc.VectorSubcoreMesh(core_axis_name="c", subcore_axis_name="s")
_sc_params = lambda: pltpu.CompilerParams(
    use_tc_tiling_on_sc=False, needs_layout_passes=False
)


def _deg_kernel():
    @functools.partial(
        pl.kernel,
        mesh=_mesh(),
        out_type=jax.ShapeDtypeStruct((2 * N_PAD,), jnp.float32),
        compiler_params=_sc_params(),
        scratch_types=[
            pltpu.VMEM((K_BLK, BLK), jnp.int32),
            pltpu.VMEM((BLK,), jnp.float32),
            pltpu.VMEM((ROWS_PER_TILE,), jnp.float32),
            pltpu.VMEM_SHARED((N_PAD,), jnp.float32),
        ],
    )
    def degk(dst_hbm, out_hbm, dst_v, ones_v, zero_v, acc_sh):
        c = lax.axis_index("c")
        s = lax.axis_index("s")
        pltpu.sync_copy(dst_hbm.at[c * 16 + s], dst_v)
        ones16 = jnp.ones((16,), jnp.float32)
        zero16 = jnp.zeros((16,), jnp.float32)
        for i in range(BLK // 16):
            ones_v[pl.ds(i * 16, 16)] = ones16

        def zb(i, carry):
            zero_v[pl.ds(i * 16, 16)] = zero16
            return carry

        lax.fori_loop(0, ROWS_PER_TILE // 16, zb, 0)
        pltpu.sync_copy(zero_v, acc_sh.at[pl.ds(s * ROWS_PER_TILE, ROWS_PER_TILE)])
        plsc.subcore_barrier()

        def body(j, carry):
            pltpu.sync_copy(ones_v, acc_sh.at[dst_v.at[j]], add=True)
            return carry

        lax.fori_loop(0, K_BLK, body, 0)
        plsc.subcore_barrier()
        pltpu.sync_copy(
            acc_sh.at[pl.ds(s * ROWS_PER_TILE, ROWS_PER_TILE)],
            out_hbm.at[pl.ds(c * N_PAD + s * ROWS_PER_TILE, ROWS_PER_TILE)],
        )

    return degk


def _agg_kernel(d, scale_in_kernel):
    extra_in = [0] if scale_in_kernel else []

    @functools.partial(
        pl.kernel,
        mesh=_mesh(),
        out_type=jax.ShapeDtypeStruct((2, N_PAD, d), jnp.float32),
        compiler_params=_sc_params(),
        scratch_types=[
            pltpu.VMEM((K_BLK, BLK), jnp.int32),
            pltpu.VMEM((K_BLK, BLK), jnp.int32),
            pltpu.VMEM((BLK, d), jnp.float32),
            pltpu.VMEM((BLK, d), jnp.float32),
            pltpu.VMEM((ROWS_PER_TILE, d), jnp.float32),
            pltpu.VMEM((ROWS_PER_TILE,), jnp.float32),
            pltpu.VMEM((ROWS_PER_TILE,), jnp.float32),
            pltpu.SemaphoreType.DMA,
            pltpu.SemaphoreType.DMA,
            pltpu.SemaphoreType.DMA,
            pltpu.SemaphoreType.DMA,
            pltpu.VMEM_SHARED((N_PAD, d), jnp.float32),
            pltpu.VMEM_SHARED((N_PAD, d), jnp.float32),
        ],
    )
    def aggk(p_hbm, src_hbm, dst_hbm, deg_hbm, out_hbm,
             src_v, dst_v, buf0, buf1, zero_v, deg_a, deg_b,
             gs0, gs1, ss0, ss1, acc_sh, p_sh):
        c = lax.axis_index("c")
        s = lax.axis_index("s")
        wid = c * 16 + s
        pltpu.sync_copy(src_hbm.at[wid], src_v)
        pltpu.sync_copy(dst_hbm.at[wid], dst_v)
        zero16 = jnp.zeros((16,), jnp.float32)

        if scale_in_kernel:
            base = s * ROWS_PER_TILE
            pltpu.sync_copy(p_hbm.at[pl.ds(base, ROWS_PER_TILE)], zero_v)
            pltpu.sync_copy(deg_hbm.at[pl.ds(base, ROWS_PER_TILE)], deg_a)
            pltpu.sync_copy(deg_hbm.at[pl.ds(N_PAD + base, ROWS_PER_TILE)], deg_b)

            def sc_body(i, carry):
                dv = deg_a[pl.ds(i * 16, 16)] + deg_b[pl.ds(i * 16, 16)] + 1.0
                bits = plsc.bitcast(dv, jnp.int32)
                y = plsc.bitcast(
                    jnp.int32(0x5F3759DF) - (bits >> 1), jnp.float32
                )
                for _ in range(3):
                    y = y * (1.5 - 0.5 * dv * y * y)
                for lane in range(16):
                    dbb = lax.gather(
                        y,
                        jnp.full((16, 1), lane, jnp.int32),
                        lax.GatherDimensionNumbers(
                            offset_dims=(),
                            collapsed_slice_dims=(0,),
                            start_index_map=(0,),
                        ),
                        (1,),
                        mode=lax.GatherScatterMode.PROMISE_IN_BOUNDS,
                    )
                    r = i * 16 + lane
                    for cc in range(d // 16):
                        zero_v[r, pl.ds(cc * 16, 16)] = (
                            zero_v[r, pl.ds(cc * 16, 16)] * dbb
                        )
                return carry

            lax.fori_loop(0, ROWS_PER_TILE // 16, sc_body, 0)
            pltpu.sync_copy(zero_v, p_sh.at[pl.ds(base, ROWS_PER_TILE)])
        else:
            pltpu.sync_copy(
                p_hbm.at[pl.ds(s * ROWS_PER_TILE, ROWS_PER_TILE)],
                p_sh.at[pl.ds(s * ROWS_PER_TILE, ROWS_PER_TILE)],
            )

        def zb(r, carry):
            for cc in range(d // 16):
                zero_v[r, pl.ds(cc * 16, 16)] = zero16
            return carry

        lax.fori_loop(0, ROWS_PER_TILE, zb, 0)
        pltpu.sync_copy(zero_v, acc_sh.at[pl.ds(s * ROWS_PER_TILE, ROWS_PER_TILE)])
        plsc.subcore_barrier()

        half = K_BLK // 2
        pltpu.async_copy(p_sh.at[src_v.at[0]], buf0, gs0)

        def body(i, carry):
            j0 = 2 * i
            pltpu.make_async_copy(p_sh.at[src_v.at[j0]], buf0, gs0).wait()

            @pl.when(i > 0)
            def _():
                pltpu.make_async_copy(buf1, acc_sh.at[dst_v.at[j0 - 1]], ss1).wait()

            pltpu.async_copy(p_sh.at[src_v.at[j0 + 1]], buf1, gs1)
            pltpu.async_copy(buf0, acc_sh.at[dst_v.at[j0]], ss0, add=True)
            pltpu.make_async_copy(p_sh.at[src_v.at[j0 + 1]], buf1, gs1).wait()
            pltpu.make_async_copy(buf0, acc_sh.at[dst_v.at[j0]], ss0).wait()

            @pl.when(i < half - 1)
            def _():
                pltpu.async_copy(p_sh.at[src_v.at[j0 + 2]], buf0, gs0)

            pltpu.async_copy(buf1, acc_sh.at[dst_v.at[j0 + 1]], ss1, add=True)
            return carry

        lax.fori_loop(0, half, body, 0)
        pltpu.make_async_copy(buf1, acc_sh.at[dst_v.at[K_BLK - 1]], ss1).wait()
        plsc.subcore_barrier()
        pltpu.sync_copy(
            acc_sh.at[pl.ds(s * ROWS_PER_TILE, ROWS_PER_TILE)],
            out_hbm.at[c, pl.ds(s * ROWS_PER_TILE, ROWS_PER_TILE)],
        )

    return aggk


def _dinv_from_flat(degf_ref):
    deg = degf_ref[pl.ds(0, N_PAD)] + degf_ref[pl.ds(N_PAD, N_PAD)] + 1.0
    return jnp.transpose(lax.rsqrt(deg).reshape(1, N_PAD), (1, 0))


def _tc_stage_b(x_ref, w1_ref, u1_ref):
    u = jnp.dot(x_ref[...], w1_ref[...], preferred_element_type=jnp.float32)
    u1_ref[pl.ds(0, N_NODES)] = u
    u1_ref[pl.ds(N_NODES, N_PAD - N_NODES)] = jnp.zeros(
        (N_PAD - N_NODES, 16), jnp.float32
    )


def _tc_stage_d(parts_ref, u1_ref, degf_ref, b1_ref, w2_ref, p2_ref):
    dinv = _dinv_from_flat(degf_ref)
    agg = parts_ref[0] + parts_ref[1] + u1_ref[...] * dinv
    h1 = jnp.maximum(agg * dinv + b1_ref[...], 0.0)
    p2 = jnp.dot(h1 * dinv, w2_ref[...], preferred_element_type=jnp.float32)
    p2_ref[pl.ds(0, N_NODES)] = p2[:N_NODES]
    p2_ref[pl.ds(N_NODES, N_PAD - N_NODES)] = jnp.zeros(
        (N_PAD - N_NODES, 32), jnp.float32
    )


def _tc_stage_f(parts_ref, p2_ref, degf_ref, b2_ref, wout_ref, bout_ref, out_ref):
    dinv = _dinv_from_flat(degf_ref)[:N_NODES]
    agg = parts_ref[0, pl.ds(0, N_NODES)] + parts_ref[1, pl.ds(0, N_NODES)]
    agg = agg + p2_ref[pl.ds(0, N_NODES)]
    h2 = jnp.maximum(agg * dinv + b2_ref[...], 0.0)
    out = jnp.dot(h2, wout_ref[...], preferred_element_type=jnp.float32)
    out_ref[...] = (out + bout_ref[...]).reshape(N_NODES // 10, 10, D_FEAT)


def kernel(x, edge_index, W1, b1, W2, b2, W_out, b_out):
    e_pad = jnp.pad(
        edge_index, ((0, 0), (0, E_PAD - N_EDGES)), constant_values=PAD_ROW
    )
    src = e_pad[0].reshape(32, K_BLK, BLK)
    dst = e_pad[1].reshape(32, K_BLK, BLK)

    deg_flat = _deg_kernel()(dst)

    u1 = pl.pallas_call(
        _tc_stage_b,
        out_shape=jax.ShapeDtypeStruct((N_PAD, 16), jnp.float32),
    )(x, W1)

    agg1 = _agg_kernel(16, True)(u1, src, dst, deg_flat)

    p2 = pl.pallas_call(
        _tc_stage_d,
        out_shape=jax.ShapeDtypeStruct((N_PAD, 32), jnp.float32),
    )(agg1, u1, deg_flat, b1, W2)

    agg2 = _agg_kernel(32, False)(p2, src, dst, deg_flat)

    return pl.pallas_call(
        _tc_stage_f,
        out_shape=jax.ShapeDtypeStruct((N_NODES // 10, 10, D_FEAT), jnp.float32),
    )(agg2, p2, deg_flat, b2, W_out, b_out)
